# baseline (speedup 1.0000x reference)
import jax
import jax.numpy as jnp
from jax.experimental import pallas as pl
from jax.experimental.pallas import tpu as pltpu

_T = 8192
_C = 1024


def _vq_body(flat_ref, embt_ref, out_ref, es_ref):
    x = flat_ref[...]
    embt = embt_ref[...]
    dd = x.shape[1]
    k = embt.shape[1]

    @pl.when(pl.program_id(0) == 0)
    def _():
        es_ref[...] = jax.lax.dot_general(
            embt * embt, jnp.ones((dd, 1), jnp.float32),
            (((0,), (0,)), ((), ())),
            precision=jax.lax.Precision.HIGHEST,
            preferred_element_type=jnp.float32)

    es = es_ref[...]
    x2 = x + x
    rn = jax.lax.dot_general(
        jnp.ones((1, dd), jnp.float32), x * x,
        (((1,), (1,)), ((), ())),
        precision=jax.lax.Precision.HIGHEST,
        preferred_element_type=jnp.float32)
    cbase = jax.lax.bitcast_convert_type(rn, jnp.int32) << 11
    kio = jax.lax.broadcasted_iota(jnp.int32, (k, 1), 0) + (1 << 30)

    for c in range(x.shape[0] // _C):
        sl = slice(c * _C, (c + 1) * _C)
        s2c = jax.lax.dot_general(
            embt, x2[sl, :], (((0,), (1,)), ((), ())),
            preferred_element_type=jnp.float32)
        dc = (rn[:, sl] - s2c) + es
        combc = ((jax.lax.bitcast_convert_type(dc, jnp.int32) << 11)
                 - cbase[:, sl]) + kio
        mnc = jnp.min(jax.lax.bitcast_convert_type(combc, jnp.float32), axis=0)
        out_ref[0, 0, sl] = jax.lax.bitcast_convert_type(mnc, jnp.int32) & 2047


def kernel(z_e_x, emb):
    B, D, H, W = z_e_x.shape
    K = emb.shape[0]
    flat = jnp.transpose(z_e_x, (0, 2, 3, 1)).reshape(-1, D)
    embt = jnp.transpose(emb)
    N = flat.shape[0]
    nb = N // _T
    out = pl.pallas_call(
        _vq_body,
        grid=(nb,),
        in_specs=[
            pl.BlockSpec((_T, D), lambda i: (i, 0)),
            pl.BlockSpec((D, K), lambda i: (0, 0)),
        ],
        out_specs=pl.BlockSpec((1, 1, _T), lambda i: (i, 0, 0)),
        out_shape=jax.ShapeDtypeStruct((nb, 1, _T), jnp.int32),
        scratch_shapes=[pltpu.VMEM((K, 1), jnp.float32)],
    )(flat, embt)
    return out.reshape(B, H, W)


# speedup vs baseline: 1.0080x; 1.0080x over previous
import jax
import jax.numpy as jnp
from jax.experimental import pallas as pl
from jax.experimental.pallas import tpu as pltpu

_T = 2048
_C = 512


def _vq_body(flat_ref, embt_ref, out_ref, es_ref):
    x = flat_ref[...]
    embt = embt_ref[...]
    dd = x.shape[1]
    k = embt.shape[1]

    @pl.when(pl.program_id(0) == 0)
    def _():
        es_ref[...] = jax.lax.dot_general(
            embt * embt, jnp.ones((dd, 1), jnp.float32),
            (((0,), (0,)), ((), ())),
            precision=jax.lax.Precision.HIGHEST,
            preferred_element_type=jnp.float32)

    es = es_ref[...]
    x2 = x + x
    rn = jax.lax.dot_general(
        jnp.ones((1, dd), jnp.float32), x * x,
        (((1,), (1,)), ((), ())),
        precision=jax.lax.Precision.HIGHEST,
        preferred_element_type=jnp.float32)
    cbase = jax.lax.bitcast_convert_type(rn, jnp.int32) << 11
    kio = jax.lax.broadcasted_iota(jnp.int32, (k, 1), 0) + (1 << 30)

    for c in range(x.shape[0] // _C):
        sl = slice(c * _C, (c + 1) * _C)
        s2c = jax.lax.dot_general(
            embt, x2[sl, :], (((0,), (1,)), ((), ())),
            preferred_element_type=jnp.float32)
        dc = (rn[:, sl] - s2c) + es
        combc = ((jax.lax.bitcast_convert_type(dc, jnp.int32) << 11)
                 - cbase[:, sl]) + kio
        mnc = jnp.min(jax.lax.bitcast_convert_type(combc, jnp.float32), axis=0)
        out_ref[0, 0, sl] = jax.lax.bitcast_convert_type(mnc, jnp.int32) & 2047


def kernel(z_e_x, emb):
    B, D, H, W = z_e_x.shape
    K = emb.shape[0]
    flat = jnp.transpose(z_e_x, (0, 2, 3, 1)).reshape(-1, D)
    embt = jnp.transpose(emb)
    N = flat.shape[0]
    nb = N // _T
    out = pl.pallas_call(
        _vq_body,
        grid=(nb,),
        in_specs=[
            pl.BlockSpec((_T, D), lambda i: (i, 0)),
            pl.BlockSpec((D, K), lambda i: (0, 0)),
        ],
        out_specs=pl.BlockSpec((1, 1, _T), lambda i: (i, 0, 0)),
        out_shape=jax.ShapeDtypeStruct((nb, 1, _T), jnp.int32),
        scratch_shapes=[pltpu.VMEM((K, 1), jnp.float32)],
    )(flat, embt)
    return out.reshape(B, H, W)
